# initial kernel scaffold (unmeasured)
import jax
import jax.numpy as jnp
from jax import lax
from jax.experimental import pallas as pl
from jax.experimental.pallas import tpu as pltpu

B, S, H, Dh, Dr = 2, 512, 16, 128, 32
D = 2048
DC = 128
SCALE = (Dh + Dr) ** -0.5

_ANY = getattr(pltpu, "ANY", None) or pltpu.MemorySpace.ANY
_CompilerParams = getattr(pltpu, "CompilerParams", None) or pltpu.TPUCompilerParams


def kernel(x, Wdkv, Wuk, Wuv, Wq, Wqr, Wkr, Wo):
    def body(
        x_ref, Wdkv_ref, Wuk_ref, Wuv_ref, Wq_ref, Wqr_ref, Wkr_ref, Wo_ref,
        out_ref,
        xbf, kbuf, vbuf, qrbuf, krbuf,
        c_send, c_recv, wk_send, wk_recv, wv_send, wv_recv,
        wqc, woc,
        send_sems, recv_sems, copy_sems,
    ):
        my_x = lax.axis_index("x")
        my_y = lax.axis_index("y")
        my_z = lax.axis_index("z")
        peer = (my_x, 1 - my_y, my_z)

        barrier = pltpu.get_barrier_semaphore()
        pl.semaphore_signal(
            barrier, inc=1, device_id=peer, device_id_type=pl.DeviceIdType.MESH
        )
        pl.semaphore_wait(barrier, 1)

        wk_send[...] = Wuk_ref[...].astype(jnp.bfloat16)
        wv_send[...] = Wuv_ref[...].astype(jnp.bfloat16)
        rdma_wk = pltpu.make_async_remote_copy(
            src_ref=wk_send, dst_ref=wk_recv,
            send_sem=send_sems.at[0], recv_sem=recv_sems.at[0],
            device_id=peer, device_id_type=pl.DeviceIdType.MESH,
        )
        rdma_wk.start()
        rdma_wv = pltpu.make_async_remote_copy(
            src_ref=wv_send, dst_ref=wv_recv,
            send_sem=send_sems.at[1], recv_sem=recv_sems.at[1],
            device_id=peer, device_id_type=pl.DeviceIdType.MESH,
        )
        rdma_wv.start()

        xbf[...] = x_ref[...].astype(jnp.bfloat16)
        wdkv_bf = Wdkv_ref[...].astype(jnp.bfloat16)
        for b in range(B):
            c_send[b, :, :] = jnp.dot(
                xbf[b, :, :], wdkv_bf, preferred_element_type=jnp.float32
            ).astype(jnp.bfloat16)
        rdma_c = pltpu.make_async_remote_copy(
            src_ref=c_send, dst_ref=c_recv,
            send_sem=send_sems.at[2], recv_sem=recv_sems.at[2],
            device_id=peer, device_id_type=pl.DeviceIdType.MESH,
        )
        rdma_c.start()

        wqr_bf = Wqr_ref[...].astype(jnp.bfloat16)
        wkr_bf = Wkr_ref[...].astype(jnp.bfloat16)
        for b in range(B):
            qrbuf[b, :, :] = jnp.dot(
                xbf[b, :, :], wqr_bf, preferred_element_type=jnp.float32
            ).astype(jnp.bfloat16)
            krbuf[b, :, :] = jnp.dot(
                xbf[b, :, :], wkr_bf, preferred_element_type=jnp.float32
            ).astype(jnp.bfloat16)
        for b in range(B):
            kbuf[b, :, :] = jnp.dot(
                c_send[b, :, :], wk_send[...], preferred_element_type=jnp.float32
            ).astype(jnp.bfloat16)
            vbuf[b, :, :] = jnp.dot(
                c_send[b, :, :], wv_send[...], preferred_element_type=jnp.float32
            ).astype(jnp.bfloat16)

        rdma_wk.wait()
        rdma_wv.wait()
        rdma_c.wait()
        for b in range(B):
            kbuf[b, :, :] = (
                kbuf[b, :, :].astype(jnp.float32)
                + jnp.dot(
                    c_recv[b, :, :], wk_recv[...],
                    preferred_element_type=jnp.float32,
                )
            ).astype(jnp.bfloat16)
            vbuf[b, :, :] = (
                vbuf[b, :, :].astype(jnp.float32)
                + jnp.dot(
                    c_recv[b, :, :], wv_recv[...],
                    preferred_element_type=jnp.float32,
                )
            ).astype(jnp.bfloat16)

        nt = (((1,), (1,)), ((), ()))
        for h in range(H):
            cp_q = pltpu.make_async_copy(
                Wq_ref.at[:, h * Dh:(h + 1) * Dh], wqc, copy_sems.at[0]
            )
            cp_o = pltpu.make_async_copy(
                Wo_ref.at[h * Dh:(h + 1) * Dh, :], woc, copy_sems.at[1]
            )
            cp_q.start()
            cp_o.start()
            cp_q.wait()
            cp_o.wait()
            wq_bf = wqc[...].astype(jnp.bfloat16)
            wo_bf = woc[...].astype(jnp.bfloat16)
            for b in range(B):
                q_h = jnp.dot(
                    xbf[b, :, :], wq_bf, preferred_element_type=jnp.float32
                ).astype(jnp.bfloat16)
                k_h = kbuf[b, :, h * Dh:(h + 1) * Dh]
                v_h = vbuf[b, :, h * Dh:(h + 1) * Dh]
                qr_h = qrbuf[b, :, h * Dr:(h + 1) * Dr]
                s = lax.dot_general(
                    q_h, k_h, nt, preferred_element_type=jnp.float32
                )
                s = s + lax.dot_general(
                    qr_h, krbuf[b, :, :], nt, preferred_element_type=jnp.float32
                )
                s = s * SCALE
                m = jnp.max(s, axis=1, keepdims=True)
                p = jnp.exp(s - m)
                p = p / jnp.sum(p, axis=1, keepdims=True)
                o_h = jnp.dot(
                    p.astype(jnp.bfloat16), v_h, preferred_element_type=jnp.float32
                )
                contrib = jnp.dot(
                    o_h.astype(jnp.bfloat16), wo_bf,
                    preferred_element_type=jnp.float32,
                )
                if h == 0:
                    out_ref[b, :, :] = contrib
                else:
                    out_ref[b, :, :] = out_ref[b, :, :] + contrib

    vmem = pl.BlockSpec(memory_space=pltpu.VMEM)
    hbm = pl.BlockSpec(memory_space=_ANY)
    return pl.pallas_call(
        body,
        out_shape=jax.ShapeDtypeStruct((B, S, D), jnp.float32),
        in_specs=[vmem, vmem, vmem, vmem, hbm, vmem, vmem, hbm],
        out_specs=vmem,
        scratch_shapes=[
            pltpu.VMEM((B, S, D), jnp.bfloat16),
            pltpu.VMEM((B, S, D), jnp.bfloat16),
            pltpu.VMEM((B, S, D), jnp.bfloat16),
            pltpu.VMEM((B, S, H * Dr), jnp.bfloat16),
            pltpu.VMEM((B, S, Dr), jnp.bfloat16),
            pltpu.VMEM((B, S, DC), jnp.bfloat16),
            pltpu.VMEM((B, S, DC), jnp.bfloat16),
            pltpu.VMEM((DC, D), jnp.bfloat16),
            pltpu.VMEM((DC, D), jnp.bfloat16),
            pltpu.VMEM((DC, D), jnp.bfloat16),
            pltpu.VMEM((DC, D), jnp.bfloat16),
            pltpu.VMEM((D, Dh), jnp.float32),
            pltpu.VMEM((Dh, D), jnp.float32),
            pltpu.SemaphoreType.DMA((3,)),
            pltpu.SemaphoreType.DMA((3,)),
            pltpu.SemaphoreType.DMA((2,)),
        ],
        compiler_params=_CompilerParams(collective_id=0),
    )(x, Wdkv, Wuk, Wuv, Wq, Wqr, Wkr, Wo)


# baseline (device time: 143578 ns/iter reference)
import jax
import jax.numpy as jnp
from jax import lax
from jax.experimental import pallas as pl
from jax.experimental.pallas import tpu as pltpu

B, S, H, Dh, Dr = 2, 512, 16, 128, 32
D = 2048
DC = 128
SCALE = (Dh + Dr) ** -0.5

_ANY = pl.ANY
_CompilerParams = getattr(pltpu, "CompilerParams", None) or pltpu.TPUCompilerParams


def kernel(x, Wdkv, Wuk, Wuv, Wq, Wqr, Wkr, Wo):
    def body(
        x_ref, Wdkv_ref, Wuk_ref, Wuv_ref, Wq_ref, Wqr_ref, Wkr_ref, Wo_ref,
        out_ref,
        xbf, kbuf, vbuf, qrbuf, krbuf,
        c_send, c_recv, wk_send, wk_recv, wv_send, wv_recv,
        wqc, woc,
        send_sems, recv_sems, copy_sems,
    ):
        my_x = lax.axis_index("x")
        my_y = lax.axis_index("y")
        my_z = lax.axis_index("z")
        peer = (my_x, 1 - my_y, my_z)

        barrier = pltpu.get_barrier_semaphore()
        pl.semaphore_signal(
            barrier, inc=1, device_id=peer, device_id_type=pl.DeviceIdType.MESH
        )
        pl.semaphore_wait(barrier, 1)

        wk_send[...] = Wuk_ref[...].astype(jnp.bfloat16)
        wv_send[...] = Wuv_ref[...].astype(jnp.bfloat16)
        rdma_wk = pltpu.make_async_remote_copy(
            src_ref=wk_send, dst_ref=wk_recv,
            send_sem=send_sems.at[0], recv_sem=recv_sems.at[0],
            device_id=peer, device_id_type=pl.DeviceIdType.MESH,
        )
        rdma_wk.start()
        rdma_wv = pltpu.make_async_remote_copy(
            src_ref=wv_send, dst_ref=wv_recv,
            send_sem=send_sems.at[1], recv_sem=recv_sems.at[1],
            device_id=peer, device_id_type=pl.DeviceIdType.MESH,
        )
        rdma_wv.start()

        xbf[...] = x_ref[...].astype(jnp.bfloat16)
        wdkv_bf = Wdkv_ref[...].astype(jnp.bfloat16)
        for b in range(B):
            c_send[b, :, :] = jnp.dot(
                xbf[b, :, :], wdkv_bf, preferred_element_type=jnp.float32
            ).astype(jnp.bfloat16)
        rdma_c = pltpu.make_async_remote_copy(
            src_ref=c_send, dst_ref=c_recv,
            send_sem=send_sems.at[2], recv_sem=recv_sems.at[2],
            device_id=peer, device_id_type=pl.DeviceIdType.MESH,
        )
        rdma_c.start()

        wqr_bf = Wqr_ref[...].astype(jnp.bfloat16)
        wkr_bf = Wkr_ref[...].astype(jnp.bfloat16)
        for b in range(B):
            qrbuf[b, :, :] = jnp.dot(
                xbf[b, :, :], wqr_bf, preferred_element_type=jnp.float32
            ).astype(jnp.bfloat16)
            krbuf[b, :, :] = jnp.dot(
                xbf[b, :, :], wkr_bf, preferred_element_type=jnp.float32
            ).astype(jnp.bfloat16)
        for b in range(B):
            kbuf[b, :, :] = jnp.dot(
                c_send[b, :, :], wk_send[...], preferred_element_type=jnp.float32
            ).astype(jnp.bfloat16)
            vbuf[b, :, :] = jnp.dot(
                c_send[b, :, :], wv_send[...], preferred_element_type=jnp.float32
            ).astype(jnp.bfloat16)

        rdma_wk.wait()
        rdma_wv.wait()
        rdma_c.wait()
        for b in range(B):
            kbuf[b, :, :] = (
                kbuf[b, :, :].astype(jnp.float32)
                + jnp.dot(
                    c_recv[b, :, :], wk_recv[...],
                    preferred_element_type=jnp.float32,
                )
            ).astype(jnp.bfloat16)
            vbuf[b, :, :] = (
                vbuf[b, :, :].astype(jnp.float32)
                + jnp.dot(
                    c_recv[b, :, :], wv_recv[...],
                    preferred_element_type=jnp.float32,
                )
            ).astype(jnp.bfloat16)

        nt = (((1,), (1,)), ((), ()))
        for h in range(H):
            cp_q = pltpu.make_async_copy(
                Wq_ref.at[:, h * Dh:(h + 1) * Dh], wqc, copy_sems.at[0]
            )
            cp_o = pltpu.make_async_copy(
                Wo_ref.at[h * Dh:(h + 1) * Dh, :], woc, copy_sems.at[1]
            )
            cp_q.start()
            cp_o.start()
            cp_q.wait()
            cp_o.wait()
            wq_bf = wqc[...].astype(jnp.bfloat16)
            wo_bf = woc[...].astype(jnp.bfloat16)
            for b in range(B):
                q_h = jnp.dot(
                    xbf[b, :, :], wq_bf, preferred_element_type=jnp.float32
                ).astype(jnp.bfloat16)
                k_h = kbuf[b, :, h * Dh:(h + 1) * Dh]
                v_h = vbuf[b, :, h * Dh:(h + 1) * Dh]
                qr_h = qrbuf[b, :, h * Dr:(h + 1) * Dr]
                s = lax.dot_general(
                    q_h, k_h, nt, preferred_element_type=jnp.float32
                )
                s = s + lax.dot_general(
                    qr_h, krbuf[b, :, :], nt, preferred_element_type=jnp.float32
                )
                s = s * SCALE
                m = jnp.max(s, axis=1, keepdims=True)
                p = jnp.exp(s - m)
                p = p / jnp.sum(p, axis=1, keepdims=True)
                o_h = jnp.dot(
                    p.astype(jnp.bfloat16), v_h, preferred_element_type=jnp.float32
                )
                contrib = jnp.dot(
                    o_h.astype(jnp.bfloat16), wo_bf,
                    preferred_element_type=jnp.float32,
                )
                if h == 0:
                    out_ref[b, :, :] = contrib
                else:
                    out_ref[b, :, :] = out_ref[b, :, :] + contrib

    vmem = pl.BlockSpec(memory_space=pltpu.VMEM)
    hbm = pl.BlockSpec(memory_space=_ANY)
    return pl.pallas_call(
        body,
        out_shape=jax.ShapeDtypeStruct((B, S, D), jnp.float32),
        in_specs=[vmem, vmem, vmem, vmem, hbm, vmem, vmem, hbm],
        out_specs=vmem,
        scratch_shapes=[
            pltpu.VMEM((B, S, D), jnp.bfloat16),
            pltpu.VMEM((B, S, D), jnp.bfloat16),
            pltpu.VMEM((B, S, D), jnp.bfloat16),
            pltpu.VMEM((B, S, H * Dr), jnp.bfloat16),
            pltpu.VMEM((B, S, Dr), jnp.bfloat16),
            pltpu.VMEM((B, S, DC), jnp.bfloat16),
            pltpu.VMEM((B, S, DC), jnp.bfloat16),
            pltpu.VMEM((DC, D), jnp.bfloat16),
            pltpu.VMEM((DC, D), jnp.bfloat16),
            pltpu.VMEM((DC, D), jnp.bfloat16),
            pltpu.VMEM((DC, D), jnp.bfloat16),
            pltpu.VMEM((D, Dh), jnp.float32),
            pltpu.VMEM((Dh, D), jnp.float32),
            pltpu.SemaphoreType.DMA((3,)),
            pltpu.SemaphoreType.DMA((3,)),
            pltpu.SemaphoreType.DMA((2,)),
        ],
        compiler_params=_CompilerParams(collective_id=0),
    )(x, Wdkv, Wuk, Wuv, Wq, Wqr, Wkr, Wo)


# device time: 118214 ns/iter; 1.2146x vs baseline; 1.2146x over previous
import jax
import jax.numpy as jnp
from jax import lax
from jax.experimental import pallas as pl
from jax.experimental.pallas import tpu as pltpu

B, S, H, Dh, Dr = 2, 512, 16, 128, 32
D = 2048
DC = 128
SCALE = (Dh + Dr) ** -0.5

_CompilerParams = getattr(pltpu, "CompilerParams", None) or pltpu.TPUCompilerParams


def kernel(x, Wdkv, Wuk, Wuv, Wq, Wqr, Wkr, Wo):
    def body(
        x_ref, Wdkv_ref, Wuk_ref, Wuv_ref, Wq_ref, Wqr_ref, Wkr_ref, Wo_ref,
        out_ref,
        xbf, kbuf, vbuf, qrbuf, krbuf,
        c_send, c_recv, wk_send, wk_recv, wv_send, wv_recv,
        wqc, woc,
        send_sems, recv_sems, copy_sems,
    ):
        my_x = lax.axis_index("x")
        my_y = lax.axis_index("y")
        my_z = lax.axis_index("z")
        peer = (my_x, 1 - my_y, my_z)

        barrier = pltpu.get_barrier_semaphore()
        pl.semaphore_signal(
            barrier, inc=1, device_id=peer, device_id_type=pl.DeviceIdType.MESH
        )
        pl.semaphore_wait(barrier, 1)

        wk_send[...] = Wuk_ref[...].astype(jnp.bfloat16)
        wv_send[...] = Wuv_ref[...].astype(jnp.bfloat16)
        rdma_wk = pltpu.make_async_remote_copy(
            src_ref=wk_send, dst_ref=wk_recv,
            send_sem=send_sems.at[0], recv_sem=recv_sems.at[0],
            device_id=peer, device_id_type=pl.DeviceIdType.MESH,
        )
        rdma_wk.start()
        rdma_wv = pltpu.make_async_remote_copy(
            src_ref=wv_send, dst_ref=wv_recv,
            send_sem=send_sems.at[1], recv_sem=recv_sems.at[1],
            device_id=peer, device_id_type=pl.DeviceIdType.MESH,
        )
        rdma_wv.start()

        xbf[...] = x_ref[...].astype(jnp.bfloat16)
        wdkv_bf = Wdkv_ref[...].astype(jnp.bfloat16)
        for b in range(B):
            c_send[b, :, :] = jnp.dot(
                xbf[b, :, :], wdkv_bf, preferred_element_type=jnp.float32
            ).astype(jnp.bfloat16)
        rdma_c = pltpu.make_async_remote_copy(
            src_ref=c_send, dst_ref=c_recv,
            send_sem=send_sems.at[2], recv_sem=recv_sems.at[2],
            device_id=peer, device_id_type=pl.DeviceIdType.MESH,
        )
        rdma_c.start()

        wqr_bf = Wqr_ref[...].astype(jnp.bfloat16)
        wkr_bf = Wkr_ref[...].astype(jnp.bfloat16)
        for b in range(B):
            qrbuf[b, :, :] = jnp.dot(
                xbf[b, :, :], wqr_bf, preferred_element_type=jnp.float32
            ).astype(jnp.bfloat16)
            krbuf[b, :, :] = jnp.dot(
                xbf[b, :, :], wkr_bf, preferred_element_type=jnp.float32
            ).astype(jnp.bfloat16)
        for b in range(B):
            kbuf[b, :, :] = jnp.dot(
                c_send[b, :, :], wk_send[...], preferred_element_type=jnp.float32
            ).astype(jnp.bfloat16)
            vbuf[b, :, :] = jnp.dot(
                c_send[b, :, :], wv_send[...], preferred_element_type=jnp.float32
            ).astype(jnp.bfloat16)

        q_cps = []
        o_cps = []
        for h in range(H):
            q_cps.append(pltpu.make_async_copy(
                Wq_ref.at[:, h * Dh:(h + 1) * Dh], wqc.at[h % 2],
                copy_sems.at[h % 2],
            ))
            o_cps.append(pltpu.make_async_copy(
                Wo_ref.at[h * Dh:(h + 1) * Dh, :], woc.at[h % 2],
                copy_sems.at[2 + h % 2],
            ))
        q_cps[0].start()
        o_cps[0].start()
        q_cps[1].start()
        o_cps[1].start()

        rdma_wk.wait()
        rdma_wv.wait()
        rdma_c.wait()
        for b in range(B):
            kbuf[b, :, :] = (
                kbuf[b, :, :].astype(jnp.float32)
                + jnp.dot(
                    c_recv[b, :, :], wk_recv[...],
                    preferred_element_type=jnp.float32,
                )
            ).astype(jnp.bfloat16)
            vbuf[b, :, :] = (
                vbuf[b, :, :].astype(jnp.float32)
                + jnp.dot(
                    c_recv[b, :, :], wv_recv[...],
                    preferred_element_type=jnp.float32,
                )
            ).astype(jnp.bfloat16)

        nt = (((1,), (1,)), ((), ()))
        for h in range(H):
            q_cps[h].wait()
            o_cps[h].wait()
            wq_bf = wqc[h % 2, :, :].astype(jnp.bfloat16)
            wo_bf = woc[h % 2, :, :].astype(jnp.bfloat16)
            if h + 2 < H:
                q_cps[h + 2].start()
                o_cps[h + 2].start()
            for b in range(B):
                q_h = jnp.dot(
                    xbf[b, :, :], wq_bf, preferred_element_type=jnp.float32
                ).astype(jnp.bfloat16)
                k_h = kbuf[b, :, h * Dh:(h + 1) * Dh]
                v_h = vbuf[b, :, h * Dh:(h + 1) * Dh]
                qr_h = qrbuf[b, :, h * Dr:(h + 1) * Dr]
                s = lax.dot_general(
                    q_h, k_h, nt, preferred_element_type=jnp.float32
                )
                s = s + lax.dot_general(
                    qr_h, krbuf[b, :, :], nt, preferred_element_type=jnp.float32
                )
                s = s * SCALE
                m = jnp.max(s, axis=1, keepdims=True)
                p = jnp.exp(s - m)
                p = p * (1.0 / jnp.sum(p, axis=1, keepdims=True))
                o_h = jnp.dot(
                    p.astype(jnp.bfloat16), v_h, preferred_element_type=jnp.float32
                )
                contrib = jnp.dot(
                    o_h.astype(jnp.bfloat16), wo_bf,
                    preferred_element_type=jnp.float32,
                )
                if h == 0:
                    out_ref[b, :, :] = contrib
                else:
                    out_ref[b, :, :] = out_ref[b, :, :] + contrib

    vmem = pl.BlockSpec(memory_space=pltpu.VMEM)
    hbm = pl.BlockSpec(memory_space=pl.ANY)
    return pl.pallas_call(
        body,
        out_shape=jax.ShapeDtypeStruct((B, S, D), jnp.float32),
        in_specs=[vmem, vmem, vmem, vmem, hbm, vmem, vmem, hbm],
        out_specs=vmem,
        scratch_shapes=[
            pltpu.VMEM((B, S, D), jnp.bfloat16),
            pltpu.VMEM((B, S, D), jnp.bfloat16),
            pltpu.VMEM((B, S, D), jnp.bfloat16),
            pltpu.VMEM((B, S, H * Dr), jnp.bfloat16),
            pltpu.VMEM((B, S, Dr), jnp.bfloat16),
            pltpu.VMEM((B, S, DC), jnp.bfloat16),
            pltpu.VMEM((B, S, DC), jnp.bfloat16),
            pltpu.VMEM((DC, D), jnp.bfloat16),
            pltpu.VMEM((DC, D), jnp.bfloat16),
            pltpu.VMEM((DC, D), jnp.bfloat16),
            pltpu.VMEM((DC, D), jnp.bfloat16),
            pltpu.VMEM((2, D, Dh), jnp.float32),
            pltpu.VMEM((2, Dh, D), jnp.float32),
            pltpu.SemaphoreType.DMA((3,)),
            pltpu.SemaphoreType.DMA((3,)),
            pltpu.SemaphoreType.DMA((4,)),
        ],
        compiler_params=_CompilerParams(collective_id=0),
    )(x, Wdkv, Wuk, Wuv, Wq, Wqr, Wkr, Wo)
